# baseline (speedup 1.0000x reference)
import jax
import jax.numpy as jnp
from jax.experimental import pallas as pl
from jax.experimental.pallas import tpu as pltpu


def _copy_kernel(in_hbm, out_hbm, buf0, buf1, buf2, buf3, si0, si1, si2, si3,
                 so0, so1, so2, so3):
    bufs = (buf0, buf1, buf2, buf3)
    in_sems = (si0, si1, si2, si3)
    out_sems = (so0, so1, so2, so3)
    nb = in_hbm.shape[0]

    def in_copy(b):
        return pltpu.make_async_copy(in_hbm.at[b], bufs[b], in_sems[b])

    def out_copy(b):
        return pltpu.make_async_copy(bufs[b], out_hbm.at[b], out_sems[b])

    for b in range(nb):
        in_copy(b).start()
    for b in range(nb):
        in_copy(b).wait()
        out_copy(b).start()
    for b in range(nb):
        out_copy(b).wait()


def kernel(inputs_embeds, embed_table):
    del embed_table
    b, s, h = inputs_embeds.shape
    chunk = pltpu.VMEM((s, h), inputs_embeds.dtype)
    sem = pltpu.SemaphoreType.DMA
    return pl.pallas_call(
        _copy_kernel,
        in_specs=[pl.BlockSpec(memory_space=pl.ANY)],
        out_specs=pl.BlockSpec(memory_space=pl.ANY),
        out_shape=jax.ShapeDtypeStruct((b, s, h), inputs_embeds.dtype),
        scratch_shapes=[chunk, chunk, chunk, chunk, sem, sem, sem, sem,
                        sem, sem, sem, sem],
    )(inputs_embeds)


# speedup vs baseline: 1.0350x; 1.0350x over previous
import jax
import jax.numpy as jnp
from jax.experimental import pallas as pl
from jax.experimental.pallas import tpu as pltpu

_GRID = 2


def _copy_kernel(in_ref, out_ref):
    out_ref[...] = in_ref[...]


def kernel(inputs_embeds, embed_table):
    del embed_table
    b, s, h = inputs_embeds.shape
    nb = b // _GRID
    return pl.pallas_call(
        _copy_kernel,
        grid=(_GRID,),
        in_specs=[pl.BlockSpec((nb, s, h), lambda i: (i, 0, 0))],
        out_specs=pl.BlockSpec((nb, s, h), lambda i: (i, 0, 0)),
        out_shape=jax.ShapeDtypeStruct((b, s, h), inputs_embeds.dtype),
        compiler_params=pltpu.CompilerParams(
            skip_device_barrier=True,
            disable_semaphore_checks=True,
            disable_bounds_checks=True,
        ),
    )(inputs_embeds)
